# baseline (speedup 1.0000x reference)
import jax
import jax.numpy as jnp
from jax import lax
from jax.experimental import pallas as pl
from jax.experimental.pallas import tpu as pltpu
from jax.experimental.pallas import tpu_sc as plsc

_NUM_EMB = 1000000
_DIM = 64
_BATCH = 4096
_SEQ = 200

_NC = 2
_NS = 16
_NW = _NC * _NS
_BPW = _BATCH // _NW

_SLICES = ((0, 56), (56, 48), (104, 48), (152, 48))


def _make_body(off, width):
  def _body(idx_hbm, table_hbm, out_hbm, idx_v, rows_a, rows_b,
            sem_ga, sem_gb, sem_sa, sem_sb):
    wid = lax.axis_index("s") * _NC + lax.axis_index("c")
    b0 = wid * _BPW
    pltpu.sync_copy(idx_hbm.at[pl.ds(b0, _BPW), pl.ds(off, width)], idx_v)

    def wait_store(rows, sem):
      pltpu.make_async_copy(rows, out_hbm.at[b0], sem).wait()

    @pl.loop(0, _BPW // 2)
    def _pair(i):
      @pl.when(i > 0)
      def _():
        wait_store(rows_a, sem_sa)
        wait_store(rows_b, sem_sb)
      ga = pltpu.async_copy(table_hbm.at[idx_v.at[2 * i]], rows_a, sem_ga)
      gb = pltpu.async_copy(table_hbm.at[idx_v.at[2 * i + 1]], rows_b, sem_gb)
      ga.wait()
      pltpu.async_copy(rows_a, out_hbm.at[b0 + 2 * i], sem_sa)
      gb.wait()
      pltpu.async_copy(rows_b, out_hbm.at[b0 + 2 * i + 1], sem_sb)

    wait_store(rows_a, sem_sa)
    wait_store(rows_b, sem_sb)

  return _body


def kernel(x, weight):
  mesh = plsc.VectorSubcoreMesh(
      core_axis_name="c", subcore_axis_name="s",
      num_cores=_NC, num_subcores=_NS)
  xi = x.astype(jnp.int32)
  outs = []
  for off, width in _SLICES:
    out = pl.kernel(
        _make_body(off, width),
        out_type=jax.ShapeDtypeStruct((_BATCH, width, _DIM), jnp.float32),
        mesh=mesh,
        scratch_types=[
            pltpu.VMEM((_BPW, width), jnp.int32),
            pltpu.VMEM((width, _DIM), jnp.float32),
            pltpu.VMEM((width, _DIM), jnp.float32),
            pltpu.SemaphoreType.DMA,
            pltpu.SemaphoreType.DMA,
            pltpu.SemaphoreType.DMA,
            pltpu.SemaphoreType.DMA,
        ],
        compiler_params=pltpu.CompilerParams(use_tc_tiling_on_sc=False),
    )(xi, weight)
    outs.append(out)
  return jnp.concatenate(outs, axis=1)
